# baseline (speedup 1.0000x reference)
import functools

import jax
import jax.numpy as jnp
from jax import lax
from jax.experimental import pallas as pl
from jax.experimental.pallas import tpu as pltpu
from jax.experimental.pallas import tpu_sc as plsc

NC, NS, L = 2, 16, 16
NW = NC * NS
B, N, S = 64, 128, 2
HW = 128 * 128
BPW = B // NW
PW = BPW * N
PWI = PW * S
NCH = PWI // L
GW = 128


def _sc_gather(out_flat, ind_flat, mask_flat):
    mesh = plsc.VectorSubcoreMesh(
        core_axis_name="c", subcore_axis_name="s",
        num_cores=NC, num_subcores=NS)

    @functools.partial(
        pl.kernel,
        out_type=[jax.ShapeDtypeStruct((B * N * S,), jnp.float32),
                  jax.ShapeDtypeStruct((B * N * S,), jnp.float32)],
        mesh=mesh,
        scratch_types=[
            pltpu.VMEM((PWI,), jnp.int32),
            pltpu.VMEM((PWI,), jnp.int32),
            pltpu.VMEM((PWI,), jnp.float32),
            pltpu.VMEM((PWI,), jnp.int32),
            pltpu.VMEM((PWI,), jnp.float32),
            pltpu.SemaphoreType.DMA,
            pltpu.SemaphoreType.DMA,
            pltpu.SemaphoreType.DMA,
        ],
    )
    def k(out_hbm, ind_hbm, mask_hbm, pred_hbm, me_hbm,
          dup_v, indd_v, me_v, pix_v, p_v, sem, wsem, msem):
        wid = lax.axis_index("s") * NC + lax.axis_index("c")
        b0 = wid * BPW
        p0 = wid * PW
        q0 = wid * PWI
        iota = lax.iota(jnp.int32, L)
        half = iota >> 1
        parity = iota & 1
        for i in range(NCH):
            dup_v[pl.ds(i * L, L)] = half + (p0 + i * (L // 2))
        gi, gm = [], []
        for w in range(PWI // GW):
            win = pl.ds(w * GW, GW)
            gi.append(pltpu.async_copy(
                ind_hbm.at[dup_v.at[win]], indd_v.at[win], sem))
            gm.append(pltpu.async_copy(
                mask_hbm.at[dup_v.at[win]], me_v.at[win], msem))
        for g in gm:
            g.wait()
        wm = pltpu.async_copy(me_v, me_hbm.at[pl.ds(q0, PWI)], wsem)
        for g in gi:
            g.wait()
        cpb = NCH // BPW
        for i in range(NCH):
            sl = pl.ds(i * L, L)
            base = (b0 + i // cpb) * (S * HW)
            pix_v[sl] = indd_v[sl] + (parity * HW + base)
        ps = []
        for w in range(PWI // GW):
            win = pl.ds(w * GW, GW)
            ps.append(pltpu.async_copy(
                out_hbm.at[pix_v.at[win]], p_v.at[win], sem))
        for g in ps:
            g.wait()
        wp = pltpu.async_copy(p_v, pred_hbm.at[pl.ds(q0, PWI)], wsem)
        wm.wait()
        wp.wait()

    return k(out_flat, ind_flat, mask_flat)


def _tc_loss(pred2, me2, tgt2):
    def k(p_ref, me_ref, t_ref, o_ref):
        p = p_ref[...]
        me = me_ref[...]
        t = t_ref[...]
        num = jnp.sum(jnp.abs(p * me - t * me), keepdims=True)
        den = jnp.sum(me, keepdims=True) * 0.5 + 0.0001
        o_ref[...] = num / den

    return pl.pallas_call(
        k, out_shape=jax.ShapeDtypeStruct((1, 1), jnp.float32),
    )(pred2, me2, tgt2)


def kernel(out, target, ind, mask):
    pred, me = _sc_gather(out.reshape(-1), ind.reshape(-1), mask.reshape(-1))
    r = _tc_loss(pred.reshape(B, N * S), me.reshape(B, N * S),
                 target.reshape(B, N * S))
    return r.reshape(())


# speedup vs baseline: 1.1151x; 1.1151x over previous
import functools

import jax
import jax.numpy as jnp
from jax import lax
from jax.experimental import pallas as pl
from jax.experimental.pallas import tpu as pltpu
from jax.experimental.pallas import tpu_sc as plsc

NC, NS, L = 2, 16, 16
NW = NC * NS
B, N, S = 64, 128, 2
HW = 128 * 128
BPW = B // NW
PW = BPW * N
PWI = PW * S
NCH = PWI // L
GW = 128


def _sc_gather(out_flat, ind_flat, mask_flat):
    mesh = plsc.VectorSubcoreMesh(
        core_axis_name="c", subcore_axis_name="s",
        num_cores=NC, num_subcores=NS)

    @functools.partial(
        pl.kernel,
        out_type=[jax.ShapeDtypeStruct((B * N * S,), jnp.float32),
                  jax.ShapeDtypeStruct((B * N * S,), jnp.float32)],
        mesh=mesh,
        scratch_types=[
            pltpu.VMEM((PWI,), jnp.int32),
            pltpu.VMEM((PWI,), jnp.int32),
            pltpu.VMEM((PWI,), jnp.float32),
            pltpu.VMEM((PWI,), jnp.int32),
            pltpu.VMEM((PWI,), jnp.float32),
            pltpu.SemaphoreType.DMA,
            pltpu.SemaphoreType.DMA,
            pltpu.SemaphoreType.DMA,
        ],
    )
    def k(out_hbm, ind_hbm, mask_hbm, pred_hbm, me_hbm,
          dup_v, indd_v, me_v, pix_v, p_v, sem, wsem, msem):
        wid = lax.axis_index("s") * NC + lax.axis_index("c")
        b0 = wid * BPW
        p0 = wid * PW
        q0 = wid * PWI
        iota = lax.iota(jnp.int32, L)
        half = iota >> 1
        parity = iota & 1
        for i in range(NCH):
            dup_v[pl.ds(i * L, L)] = half + (p0 + i * (L // 2))
        gi, gm = [], []
        for w in range(PWI // GW):
            win = pl.ds(w * GW, GW)
            gi.append(pltpu.async_copy(
                ind_hbm.at[dup_v.at[win]], indd_v.at[win], sem))
            gm.append(pltpu.async_copy(
                mask_hbm.at[dup_v.at[win]], me_v.at[win], msem))
        for g in gm:
            g.wait()
        wm = pltpu.async_copy(me_v, me_hbm.at[pl.ds(q0, PWI)], wsem)
        for g in gi:
            g.wait()
        cpb = NCH // BPW
        for i in range(NCH):
            sl = pl.ds(i * L, L)
            base = (b0 + i // cpb) * (S * HW)
            pix_v[sl] = indd_v[sl] + (parity * HW + base)
        ps = []
        for w in range(PWI // GW):
            win = pl.ds(w * GW, GW)
            ps.append(pltpu.async_copy(
                out_hbm.at[pix_v.at[win]], p_v.at[win], sem))
        for g in ps:
            g.wait()
        wp = pltpu.async_copy(p_v, pred_hbm.at[pl.ds(q0, PWI)], wsem)
        wm.wait()
        wp.wait()

    return k(out_flat, ind_flat, mask_flat)


def _tc_loss(pred2, me2, tgt2):
    def k(p_ref, me_ref, t_ref, o_ref):
        p = p_ref[...]
        me = me_ref[...]
        t = t_ref[...]
        num = jnp.sum(jnp.abs(p * me - t * me), keepdims=True)
        den = jnp.sum(me, keepdims=True) * 0.5 + 0.0001
        o_ref[...] = num / den

    return pl.pallas_call(
        k, out_shape=jax.ShapeDtypeStruct((1, 1), jnp.float32),
    )(pred2, me2, tgt2)


def kernel(out, target, ind, mask):
    pred, me = _sc_gather(out.reshape(-1), ind.reshape(-1), mask.reshape(-1))
    r = _tc_loss(pred.reshape(N, N), me.reshape(N, N), target.reshape(N, N))
    return r.reshape(())
